# baseline (speedup 1.0000x reference)
import functools

import jax
import jax.numpy as jnp
from jax import lax
from jax.experimental import pallas as pl
from jax.experimental.pallas import tpu as pltpu
from jax.experimental.pallas import tpu_sc as plsc

VOCAB_N = 1000000
EMBED_N = 32
ALPHA_C = 0.75
X_MAX_C = 100.0
B_N = 4096

_NC = 2
_NS = 16
_NW = _NC * _NS
_BPW = B_N // _NW


def _make_gather():
    mesh = plsc.VectorSubcoreMesh(core_axis_name="c", subcore_axis_name="s")

    @functools.partial(
        pl.kernel,
        mesh=mesh,
        compiler_params=pltpu.CompilerParams(use_tc_tiling_on_sc=False),
        out_type=[
            jax.ShapeDtypeStruct((B_N, EMBED_N), jnp.float32),
            jax.ShapeDtypeStruct((B_N, EMBED_N), jnp.float32),
            jax.ShapeDtypeStruct((B_N, 1), jnp.float32),
            jax.ShapeDtypeStruct((B_N, 1), jnp.float32),
        ],
        scratch_types=[
            pltpu.VMEM((_BPW,), jnp.int32),
            pltpu.VMEM((_BPW,), jnp.int32),
            pltpu.VMEM((_BPW, EMBED_N), jnp.float32),
            pltpu.VMEM((_BPW, EMBED_N), jnp.float32),
            pltpu.VMEM((_BPW, 1), jnp.float32),
            pltpu.VMEM((_BPW, 1), jnp.float32),
            pltpu.SemaphoreType.DMA,
        ],
    )
    def gather_k(wdata_hbm, vdata_hbm, w_embed_hbm, v_embed_hbm,
                 w_bias_hbm, v_bias_hbm,
                 wrows_out, vrows_out, wb_out, vb_out,
                 widx_v, vidx_v, wrows_v, vrows_v, wb_v, vb_v, sem):
        wid = lax.axis_index("s") * _NC + lax.axis_index("c")
        base = wid * _BPW
        pltpu.sync_copy(wdata_hbm.at[pl.ds(base, _BPW)], widx_v)
        pltpu.sync_copy(vdata_hbm.at[pl.ds(base, _BPW)], vidx_v)
        cw = pltpu.async_copy(w_embed_hbm.at[widx_v], wrows_v, sem)
        cv = pltpu.async_copy(v_embed_hbm.at[vidx_v], vrows_v, sem)
        cwb = pltpu.async_copy(w_bias_hbm.at[widx_v], wb_v, sem)
        cvb = pltpu.async_copy(v_bias_hbm.at[vidx_v], vb_v, sem)
        cw.wait()
        cv.wait()
        cwb.wait()
        cvb.wait()
        pltpu.sync_copy(wrows_v, wrows_out.at[pl.ds(base, _BPW)])
        pltpu.sync_copy(vrows_v, vrows_out.at[pl.ds(base, _BPW)])
        pltpu.sync_copy(wb_v, wb_out.at[pl.ds(base, _BPW)])
        pltpu.sync_copy(vb_v, vb_out.at[pl.ds(base, _BPW)])

    return gather_k


_gather = _make_gather()


def _loss_body(wrows, vrows, wb, vb, lab, out_ref):
    w = wrows[...]
    v = vrows[...]
    s = jnp.sum(w * v, axis=1, keepdims=True)
    b = wb[...] + vb[...]
    l = lab[...]
    wt = jnp.minimum(jnp.power(l * (1.0 / X_MAX_C), ALPHA_C), 1.0)
    c = s - jnp.log(l)
    s_wt = jnp.sum(wt)
    s_b = jnp.sum(b)
    s_b2 = jnp.sum(b * b)
    wtc = wt * c
    s_wtc = jnp.sum(wtc)
    s_wtc2 = jnp.sum(wtc * c)
    bsz = jnp.float32(B_N)
    out_ref[0, 0] = (s_wt * s_b2 + 2.0 * s_b * s_wtc + bsz * s_wtc2) / (bsz * bsz)


def _loss_tc(wrows, vrows, wb, vb, labels2d):
    return pl.pallas_call(
        _loss_body,
        out_shape=jax.ShapeDtypeStruct((1, 1), jnp.float32),
        out_specs=pl.BlockSpec(memory_space=pltpu.SMEM),
    )(wrows, vrows, wb, vb, labels2d)


def kernel(wdata, vdata, labels, w_embed, v_embed, w_bias, v_bias):
    wrows, vrows, wb, vb = _gather(wdata, vdata, w_embed, v_embed,
                                   w_bias, v_bias)
    out = _loss_tc(wrows, vrows, wb, vb, labels.reshape(B_N, 1))
    return out[0, 0]


# speedup vs baseline: 36.5872x; 36.5872x over previous
import functools

import jax
import jax.numpy as jnp
from jax import lax
from jax.experimental import pallas as pl
from jax.experimental.pallas import tpu as pltpu
from jax.experimental.pallas import tpu_sc as plsc

VOCAB_N = 1000000
EMBED_N = 32
ALPHA_C = 0.75
X_MAX_C = 100.0
B_N = 4096

_NC = 2
_NS = 16
_NW = _NC * _NS
_BPW = B_N // _NW
_NBUF = 8


def _make_sc_gather():
    mesh = plsc.VectorSubcoreMesh(core_axis_name="c", subcore_axis_name="s")

    @functools.partial(
        pl.kernel,
        mesh=mesh,
        compiler_params=pltpu.CompilerParams(use_tc_tiling_on_sc=True),
        out_type=[
            jax.ShapeDtypeStruct((B_N,), jnp.float32),
            jax.ShapeDtypeStruct((B_N,), jnp.float32),
            jax.ShapeDtypeStruct((B_N,), jnp.float32),
        ],
        scratch_types=[
            pltpu.SMEM((_BPW,), jnp.int32),
            pltpu.SMEM((_BPW,), jnp.int32),
            pltpu.VMEM((_BPW,), jnp.int32),
            pltpu.VMEM((_BPW,), jnp.int32),
            pltpu.VMEM((_NBUF, EMBED_N, 128), jnp.float32),
            pltpu.VMEM((_NBUF, EMBED_N, 128), jnp.float32),
            pltpu.VMEM((_NBUF, 1, 128), jnp.float32),
            pltpu.VMEM((_NBUF, 1, 128), jnp.float32),
            pltpu.VMEM((_BPW,), jnp.float32),
            pltpu.VMEM((_BPW,), jnp.float32),
            pltpu.VMEM((_BPW,), jnp.float32),
            pltpu.SemaphoreType.DMA,
            pltpu.SemaphoreType.DMA,
            pltpu.SemaphoreType.DMA,
            pltpu.SemaphoreType.DMA,
        ],
    )
    def gather_k(wet_hbm, vet_hbm, wbt_hbm, vbt_hbm, wdata_hbm, vdata_hbm,
                 s_out, wb_out, vb_out,
                 widx_s, vidx_s, widx_v, vidx_v, wtiles_v, vtiles_v,
                 wbr_v, vbr_v, s_v, wb_v, vb_v, wsem, vsem, wbsem, vbsem):
        wid = lax.axis_index("s") * _NC + lax.axis_index("c")
        base = wid * _BPW
        pltpu.sync_copy(wdata_hbm.at[pl.ds(base, _BPW)], widx_v)
        pltpu.sync_copy(vdata_hbm.at[pl.ds(base, _BPW)], vidx_v)

        lanes16 = lax.iota(jnp.int32, 16)

        for g in range(_BPW // 16):
            wchunk = widx_v[pl.ds(g * 16, 16)]
            vchunk = vidx_v[pl.ds(g * 16, 16)]
            for lane in range(16):
                widx_s[g * 16 + lane] = wchunk[lane]
                vidx_s[g * 16 + lane] = vchunk[lane]

        def fire(k, slot):
            iw = widx_s[k]
            iv = vidx_s[k]
            w_off = pl.multiple_of((iw >> 7) * 128, 128)
            v_off = pl.multiple_of((iv >> 7) * 128, 128)
            pltpu.async_copy(wet_hbm.at[:, pl.ds(w_off, 128)],
                             wtiles_v.at[slot], wsem)
            pltpu.async_copy(vet_hbm.at[:, pl.ds(v_off, 128)],
                             vtiles_v.at[slot], vsem)
            pltpu.async_copy(wbt_hbm.at[:, pl.ds(w_off, 128)],
                             wbr_v.at[slot], wbsem)
            pltpu.async_copy(vbt_hbm.at[:, pl.ds(v_off, 128)],
                             vbr_v.at[slot], vbsem)

        for k in range(_NBUF):
            fire(k, k)

        def lane_select(vec16, value_splat, lane_pos, old):
            return jnp.where(lanes16 == lane_pos, value_splat, old)

        def body(k, carry):
            s16, wb16, vb16 = carry
            slot = k & (_NBUF - 1)
            pltpu.make_async_copy(wet_hbm.at[:, pl.ds(0, 128)],
                                  wtiles_v.at[slot], wsem).wait()
            pltpu.make_async_copy(vet_hbm.at[:, pl.ds(0, 128)],
                                  vtiles_v.at[slot], vsem).wait()
            pltpu.make_async_copy(wbt_hbm.at[:, pl.ds(0, 128)],
                                  wbr_v.at[slot], wbsem).wait()
            pltpu.make_async_copy(vbt_hbm.at[:, pl.ds(0, 128)],
                                  vbr_v.at[slot], vbsem).wait()

            iw = widx_s[k]
            iv = vidx_s[k]
            lw = iw & 127
            lv = iv & 127
            cw = pl.multiple_of((lw >> 4) * 16, 16)
            cv = pl.multiple_of((lv >> 4) * 16, 16)
            sw = lw & 15
            sv = lv & 15
            rot = (lanes16 + (sv - sw)) & 15

            def edot(e, acc):
                w16 = wtiles_v[slot, e, pl.ds(cw, 16)]
                v16 = vtiles_v[slot, e, pl.ds(cv, 16)]
                vr = jnp.take(v16, rot, axis=0)
                return acc + w16 * vr

            acc = lax.fori_loop(0, EMBED_N, edot,
                                jnp.zeros((16,), jnp.float32))
            sub_w = jnp.broadcast_to(sw, (16,))
            sub_v = jnp.broadcast_to(sv, (16,))
            s_k = jnp.take(acc, sub_w, axis=0)
            wb_k = jnp.take(wbr_v[slot, 0, pl.ds(cw, 16)], sub_w, axis=0)
            vb_k = jnp.take(vbr_v[slot, 0, pl.ds(cv, 16)], sub_v, axis=0)

            kpos = k & 15
            s16 = jnp.where(lanes16 == kpos, s_k, s16)
            wb16 = jnp.where(lanes16 == kpos, wb_k, wb16)
            vb16 = jnp.where(lanes16 == kpos, vb_k, vb16)

            @pl.when(kpos == 15)
            def _():
                g = pl.multiple_of((k >> 4) * 16, 16)
                s_v[pl.ds(g, 16)] = s16
                wb_v[pl.ds(g, 16)] = wb16
                vb_v[pl.ds(g, 16)] = vb16

            @pl.when(k + _NBUF < _BPW)
            def _():
                fire(k + _NBUF, slot)

            return s16, wb16, vb16

        z16 = jnp.zeros((16,), jnp.float32)
        lax.fori_loop(0, _BPW, body, (z16, z16, z16))

        pltpu.sync_copy(s_v, s_out.at[pl.ds(base, _BPW)])
        pltpu.sync_copy(wb_v, wb_out.at[pl.ds(base, _BPW)])
        pltpu.sync_copy(vb_v, vb_out.at[pl.ds(base, _BPW)])

    return gather_k


_sc_gather = _make_sc_gather()


def _loss_body(s_ref, wb_ref, vb_ref, lab_ref, out_ref):
    s = s_ref[...]
    b = wb_ref[...] + vb_ref[...]
    l = lab_ref[...]
    wt = jnp.minimum(jnp.power(l * (1.0 / X_MAX_C), ALPHA_C), 1.0)
    c = s - jnp.log(l)
    s_wt = jnp.sum(wt)
    s_b = jnp.sum(b)
    s_b2 = jnp.sum(b * b)
    wtc = wt * c
    s_wtc = jnp.sum(wtc)
    s_wtc2 = jnp.sum(wtc * c)
    bsz = jnp.float32(B_N)
    out_ref[0, 0] = (s_wt * s_b2 + 2.0 * s_b * s_wtc + bsz * s_wtc2) / (bsz * bsz)


def _loss_tc(s, wb, vb, labels):
    return pl.pallas_call(
        _loss_body,
        out_shape=jax.ShapeDtypeStruct((1, 1), jnp.float32),
        out_specs=pl.BlockSpec(memory_space=pltpu.SMEM),
    )(s.reshape(32, 128), wb.reshape(32, 128), vb.reshape(32, 128),
      labels.reshape(32, 128))


def kernel(wdata, vdata, labels, w_embed, v_embed, w_bias, v_bias):
    s, wb, vb = _sc_gather(w_embed.T, v_embed.T, w_bias.T, v_bias.T,
                           wdata, vdata)
    out = _loss_tc(s, wb, vb, labels)
    return out[0, 0]
